# baseline (speedup 1.0000x reference)
import functools

import jax
import jax.numpy as jnp
from jax import lax
from jax.experimental import pallas as pl
from jax.experimental.pallas import tpu as pltpu
from jax.experimental.pallas import tpu_sc as plsc

STAGE1_RATE = 0.5
STAGE2_RATE = 0.75

_NC = 2
_NS = 16
_L = 16
_NW = _NC * _NS
_G = 24


def _iota16():
    return lax.iota(jnp.int32, _L)


def _splat(x, dtype=jnp.int32):
    return jnp.full((_L,), x, dtype)


def _sc_body(pred_hbm, lab_hbm, rand_hbm, ratio_hbm, zrow_hbm, out_hbm,
             lab_v, rand_v, ratio_v, idx_u, idx_m, lab_m, cbuf, obuf,
             gsem0, gsem1, ssem0, ssem1, msem0, msem1):
    rows_per_w = lab_v.shape[0]
    n_vec = rows_per_w // _L
    wid = lax.axis_index("s") * _NC + lax.axis_index("c")
    base = wid * rows_per_w

    gsem = (gsem0, gsem1)
    ssem = (ssem0, ssem1)
    msem = (msem0, msem1)

    pltpu.sync_copy(lab_hbm.at[pl.ds(base, rows_per_w)], lab_v)
    pltpu.sync_copy(rand_hbm.at[pl.ds(base, rows_per_w)], rand_v)
    pltpu.sync_copy(ratio_hbm, ratio_v)
    pltpu.sync_copy(zrow_hbm, obuf.at[pl.ds(0, _G)])
    pltpu.sync_copy(zrow_hbm, obuf.at[pl.ds(_G, _G)])

    ratio = ratio_v[...]
    iota = _iota16()
    ones_f = jnp.full((_L,), 1.0, jnp.float32)
    zeros_f = jnp.full((_L,), 0.0, jnp.float32)
    half_m = iota < _splat(8)

    def compact(i, cnt):
        cnt_u, cnt_m = cnt
        lv = lab_v[pl.ds(i * _L, _L)]
        rv = rand_v[pl.ds(i * _L, _L)]
        m = (rv < ratio) & (lv >= _splat(0))
        mi = jnp.where(m, _splat(1), _splat(0))
        ids = _splat(base) + _splat(i * _L) + iota
        pos_u = _splat(cnt_u) + plsc.cumsum(_splat(1) - mi) - _splat(1)
        plsc.store_scatter(idx_u, [pos_u // _splat(_G), pos_u % _splat(_G)],
                           ids, mask=~m)
        pos_m = _splat(cnt_m) + plsc.cumsum(mi) - _splat(1)
        plsc.store_scatter(idx_m, [pos_m // _splat(_G), pos_m % _splat(_G)],
                           ids, mask=m)
        plsc.store_scatter(lab_m, [pos_m // _splat(_G), pos_m % _splat(_G)],
                           lv, mask=m)
        return cnt_u + jnp.sum(_splat(1) - mi), cnt_m + jnp.sum(mi)

    k_u, k_m = lax.fori_loop(0, n_vec, compact, (jnp.int32(0), jnp.int32(0)))

    def pad_tail(idx2d, k, also=None):
        rem = k % _G

        @pl.when((k > 0) & (rem != 0))
        def _():
            g = k // _G
            for off in (0, _L):
                cols = iota + _splat(off)
                msk = (cols < _splat(_G)) & (cols >= _splat(rem))
                first = plsc.load_gather(idx2d, [_splat(0), _splat(0)])
                plsc.store_scatter(idx2d, [_splat(g), cols], first, mask=msk)
                if also is not None:
                    first2 = plsc.load_gather(also, [_splat(0), _splat(0)])
                    plsc.store_scatter(also, [_splat(g), cols], first2,
                                       mask=msk)

    pad_tail(idx_u, k_u)
    pad_tail(idx_m, k_m, also=lab_m)

    q_u = (k_u + (_G - 1)) // _G
    q_m = (k_m + (_G - 1)) // _G
    q_max = jnp.maximum(q_u, q_m) + 2
    n_it = (q_max + 1) >> 1

    def onehot_cols(g):
        lv0 = plsc.load_gather(lab_m, [_splat(g), iota])
        lv1 = plsc.load_gather(lab_m, [_splat(g), iota + _splat(_L)],
                               mask=half_m)
        return jnp.clip(lv0, 0, 1023), jnp.clip(lv1, 0, 1023)

    def step(t, carry):
        for b in range(2):
            g = t * 2 + b
            cslab = cbuf.at[pl.ds(b * _G, _G)]
            oslab = obuf.at[pl.ds(b * _G, _G)]

            @pl.when((g >= 2) & (g - 2 < q_u))
            def _(g=g, b=b, cslab=cslab):
                pltpu.make_async_copy(cslab, out_hbm.at[idx_u.at[g - 2]],
                                      ssem[b]).wait()

            @pl.when(g < q_u)
            def _(g=g, b=b, cslab=cslab):
                pltpu.async_copy(pred_hbm.at[idx_u.at[g]], cslab, gsem[b])

            @pl.when((g >= 2) & (g - 2 < q_m))
            def _(g=g, b=b, oslab=oslab):
                pltpu.make_async_copy(oslab, out_hbm.at[idx_m.at[g - 2]],
                                      msem[b]).wait()
                lv0, lv1 = onehot_cols(g - 2)
                plsc.store_scatter(oslab, [iota, lv0], zeros_f)
                plsc.store_scatter(oslab, [iota + _splat(_L), lv1], zeros_f,
                                   mask=half_m)

            @pl.when(g < q_m)
            def _(g=g, b=b, oslab=oslab):
                lv0, lv1 = onehot_cols(g)
                plsc.store_scatter(oslab, [iota, lv0], ones_f)
                plsc.store_scatter(oslab, [iota + _splat(_L), lv1], ones_f,
                                   mask=half_m)
                pltpu.async_copy(oslab, out_hbm.at[idx_m.at[g]], msem[b])

            ob = 1 - b
            oc = cbuf.at[pl.ds(ob * _G, _G)]

            @pl.when((g >= 1) & (g - 1 < q_u))
            def _(g=g, ob=ob, oc=oc):
                pltpu.make_async_copy(pred_hbm.at[idx_u.at[g - 1]], oc,
                                      gsem[ob]).wait()
                pltpu.async_copy(oc, out_hbm.at[idx_u.at[g - 1]], ssem[ob])

        return carry

    lax.fori_loop(0, n_it, step, jnp.int32(0))


def kernel(obj_sem_cls_pred, obj_labels, cur_step, total_steps):
    b, n, c = obj_sem_cls_pred.shape
    rows = b * n
    rows_per_w = rows // _NW
    n_gran = (rows_per_w + _G - 1) // _G + 1
    mixup_ratio = jnp.clip(
        (total_steps * STAGE2_RATE - cur_step)
        / ((STAGE2_RATE - STAGE1_RATE) * total_steps),
        0.0,
        1.0,
    ).astype(jnp.float32)
    random_numer = jax.random.uniform(
        jax.random.key(42), (b, n), dtype=jnp.float32
    )

    pred2d = obj_sem_cls_pred.reshape(rows, c)
    lab1d = obj_labels.astype(jnp.int32).reshape(rows)
    rand1d = random_numer.reshape(rows)
    ratio16 = jnp.full((_L,), mixup_ratio, jnp.float32)
    zrow = jnp.zeros((_G, c), jnp.float32)

    mesh = plsc.VectorSubcoreMesh(
        core_axis_name="c", subcore_axis_name="s",
        num_cores=_NC, num_subcores=_NS,
    )
    run = functools.partial(
        pl.kernel,
        out_type=jax.ShapeDtypeStruct((rows, c), jnp.float32),
        mesh=mesh,
        compiler_params=pltpu.CompilerParams(needs_layout_passes=False),
        scratch_types=[
            pltpu.VMEM((rows_per_w,), jnp.int32),
            pltpu.VMEM((rows_per_w,), jnp.float32),
            pltpu.VMEM((_L,), jnp.float32),
            pltpu.VMEM((n_gran, _G), jnp.int32),
            pltpu.VMEM((n_gran, _G), jnp.int32),
            pltpu.VMEM((n_gran, _G), jnp.int32),
            pltpu.VMEM((2 * _G, c), jnp.float32),
            pltpu.VMEM((2 * _G, c), jnp.float32),
            pltpu.SemaphoreType.DMA,
            pltpu.SemaphoreType.DMA,
            pltpu.SemaphoreType.DMA,
            pltpu.SemaphoreType.DMA,
            pltpu.SemaphoreType.DMA,
            pltpu.SemaphoreType.DMA,
        ],
    )(_sc_body)
    out = run(pred2d, lab1d, rand1d, ratio16, zrow)
    return out.reshape(b, n, c)


# speedup vs baseline: 1.0024x; 1.0024x over previous
import functools

import jax
import jax.numpy as jnp
from jax import lax
from jax.experimental import pallas as pl
from jax.experimental.pallas import tpu as pltpu
from jax.experimental.pallas import tpu_sc as plsc

STAGE1_RATE = 0.5
STAGE2_RATE = 0.75

_NC = 2
_NS = 16
_L = 16
_NW = _NC * _NS
_G = 24


def _iota16():
    return lax.iota(jnp.int32, _L)


def _splat(x, dtype=jnp.int32):
    return jnp.full((_L,), x, dtype)


def _sc_body(pred_hbm, lab_hbm, rand_hbm, ratio_hbm, zrow_hbm, out_hbm,
             lab_v, rand_v, ratio_v, idx_u, idx_m, lab_m, cbuf, obuf,
             gsem0, gsem1, ssem0, ssem1, msem0, msem1):
    rows_per_w = lab_v.shape[0]
    n_vec = rows_per_w // _L
    wid = lax.axis_index("s") * _NC + lax.axis_index("c")
    base = wid * rows_per_w

    gsem = (gsem0, gsem1)
    ssem = (ssem0, ssem1)
    msem = (msem0, msem1)

    pltpu.sync_copy(lab_hbm.at[pl.ds(base, rows_per_w)], lab_v)
    pltpu.sync_copy(rand_hbm.at[pl.ds(base, rows_per_w)], rand_v)
    pltpu.sync_copy(ratio_hbm, ratio_v)
    pltpu.sync_copy(zrow_hbm, obuf.at[pl.ds(0, _G)])
    pltpu.sync_copy(zrow_hbm, obuf.at[pl.ds(_G, _G)])

    ratio = ratio_v[...]
    iota = _iota16()
    ones_f = jnp.full((_L,), 1.0, jnp.float32)
    zeros_f = jnp.full((_L,), 0.0, jnp.float32)
    half_m = iota < _splat(8)

    def compact(i, cnt):
        cnt_u, cnt_m = cnt
        lv = lab_v[pl.ds(i * _L, _L)]
        rv = rand_v[pl.ds(i * _L, _L)]
        m = (rv < ratio) & (lv >= _splat(0))
        mi = jnp.where(m, _splat(1), _splat(0))
        ids = _splat(base) + _splat(i * _L) + iota
        pos_u = _splat(cnt_u) + plsc.cumsum(_splat(1) - mi) - _splat(1)
        plsc.store_scatter(idx_u, [pos_u // _splat(_G), pos_u % _splat(_G)],
                           ids, mask=~m)
        pos_m = _splat(cnt_m) + plsc.cumsum(mi) - _splat(1)
        plsc.store_scatter(idx_m, [pos_m // _splat(_G), pos_m % _splat(_G)],
                           ids, mask=m)
        plsc.store_scatter(lab_m, [pos_m // _splat(_G), pos_m % _splat(_G)],
                           lv, mask=m)
        return cnt_u + jnp.sum(_splat(1) - mi), cnt_m + jnp.sum(mi)

    k_u, k_m = lax.fori_loop(0, n_vec, compact, (jnp.int32(0), jnp.int32(0)))

    def pad_tail(idx2d, k, also=None):
        rem = k % _G

        @pl.when((k > 0) & (rem != 0))
        def _():
            g = k // _G
            for off in (0, _L):
                cols = iota + _splat(off)
                msk = (cols < _splat(_G)) & (cols >= _splat(rem))
                first = plsc.load_gather(idx2d, [_splat(0), _splat(0)])
                plsc.store_scatter(idx2d, [_splat(g), cols], first, mask=msk)
                if also is not None:
                    first2 = plsc.load_gather(also, [_splat(0), _splat(0)])
                    plsc.store_scatter(also, [_splat(g), cols], first2,
                                       mask=msk)

    pad_tail(idx_u, k_u)
    pad_tail(idx_m, k_m, also=lab_m)

    q_u = (k_u + (_G - 1)) // _G
    q_m = (k_m + (_G - 1)) // _G
    q_max = jnp.maximum(q_u, q_m) + 2
    n_it = (q_max + 1) >> 1

    def onehot_cols(g):
        lv0 = plsc.load_gather(lab_m, [_splat(g), iota])
        lv1 = plsc.load_gather(lab_m, [_splat(g), iota + _splat(_L)],
                               mask=half_m)
        return jnp.clip(lv0, 0, 1023), jnp.clip(lv1, 0, 1023)

    def step(t, carry):
        for b in range(2):
            g = t * 2 + b
            cslab = cbuf.at[pl.ds(b * _G, _G)]
            oslab = obuf.at[pl.ds(b * _G, _G)]

            @pl.when((g >= 2) & (g - 2 < q_u))
            def _(g=g, b=b, cslab=cslab):
                pltpu.make_async_copy(cslab, out_hbm.at[idx_u.at[g - 2]],
                                      ssem[b]).wait()

            @pl.when(g < q_u)
            def _(g=g, b=b, cslab=cslab):
                pltpu.async_copy(pred_hbm.at[idx_u.at[g]], cslab, gsem[b])

            @pl.when((g >= 2) & (g - 2 < q_m))
            def _(g=g, b=b, oslab=oslab):
                pltpu.make_async_copy(oslab, out_hbm.at[idx_m.at[g - 2]],
                                      msem[b]).wait()
                lv0, lv1 = onehot_cols(g - 2)
                plsc.store_scatter(oslab, [iota, lv0], zeros_f)
                plsc.store_scatter(oslab, [iota + _splat(_L), lv1], zeros_f,
                                   mask=half_m)

            @pl.when(g < q_m)
            def _(g=g, b=b, oslab=oslab):
                lv0, lv1 = onehot_cols(g)
                plsc.store_scatter(oslab, [iota, lv0], ones_f)
                plsc.store_scatter(oslab, [iota + _splat(_L), lv1], ones_f,
                                   mask=half_m)
                pltpu.async_copy(oslab, out_hbm.at[idx_m.at[g]], msem[b])

            ob = 1 - b
            oc = cbuf.at[pl.ds(ob * _G, _G)]

            @pl.when((g >= 1) & (g - 1 < q_u))
            def _(g=g, ob=ob, oc=oc):
                pltpu.make_async_copy(pred_hbm.at[idx_u.at[g - 1]], oc,
                                      gsem[ob]).wait()
                pltpu.async_copy(oc, out_hbm.at[idx_u.at[g - 1]], ssem[ob])

        return carry

    lax.fori_loop(0, n_it, step, jnp.int32(0))


def kernel(obj_sem_cls_pred, obj_labels, cur_step, total_steps):
    b, n, c = obj_sem_cls_pred.shape
    rows = b * n
    rows_per_w = rows // _NW
    n_gran = (rows_per_w + _G - 1) // _G + 1
    mixup_ratio = jnp.clip(
        (total_steps * STAGE2_RATE - cur_step)
        / ((STAGE2_RATE - STAGE1_RATE) * total_steps),
        0.0,
        1.0,
    ).astype(jnp.float32)
    random_numer = jax.random.uniform(
        jax.random.key(42), (b, n), dtype=jnp.float32
    )

    pred2d = obj_sem_cls_pred.reshape(rows, c)
    lab1d = obj_labels.astype(jnp.int32).reshape(rows)
    rand1d = random_numer.reshape(rows)
    ratio16 = jnp.full((_L,), mixup_ratio, jnp.float32)
    zrow = jnp.zeros((_G, c), jnp.float32)

    mesh = plsc.VectorSubcoreMesh(
        core_axis_name="c", subcore_axis_name="s",
        num_cores=_NC, num_subcores=_NS,
    )
    run = functools.partial(
        pl.kernel,
        out_type=jax.ShapeDtypeStruct((rows, c), jnp.float32),
        mesh=mesh,
        compiler_params=pltpu.CompilerParams(
            needs_layout_passes=False, skip_device_barrier=True),
        scratch_types=[
            pltpu.VMEM((rows_per_w,), jnp.int32),
            pltpu.VMEM((rows_per_w,), jnp.float32),
            pltpu.VMEM((_L,), jnp.float32),
            pltpu.VMEM((n_gran, _G), jnp.int32),
            pltpu.VMEM((n_gran, _G), jnp.int32),
            pltpu.VMEM((n_gran, _G), jnp.int32),
            pltpu.VMEM((2 * _G, c), jnp.float32),
            pltpu.VMEM((2 * _G, c), jnp.float32),
            pltpu.SemaphoreType.DMA,
            pltpu.SemaphoreType.DMA,
            pltpu.SemaphoreType.DMA,
            pltpu.SemaphoreType.DMA,
            pltpu.SemaphoreType.DMA,
            pltpu.SemaphoreType.DMA,
        ],
    )(_sc_body)
    out = run(pred2d, lab1d, rand1d, ratio16, zrow)
    return out.reshape(b, n, c)


# speedup vs baseline: 1.0111x; 1.0087x over previous
import functools

import jax
import jax.numpy as jnp
from jax import lax
from jax.experimental import pallas as pl
from jax.experimental.pallas import tpu as pltpu
from jax.experimental.pallas import tpu_sc as plsc

STAGE1_RATE = 0.5
STAGE2_RATE = 0.75

_NC = 2
_NS = 16
_L = 16
_NW = _NC * _NS
_G = 16
_W = 3


def _iota16():
    return lax.iota(jnp.int32, _L)


def _splat(x, dtype=jnp.int32):
    return jnp.full((_L,), x, dtype)


def _sc_body(pred_hbm, lab_hbm, rand_hbm, ratio_hbm, zrow_hbm, out_hbm,
             lab_v, rand_v, ratio_v, idx_u, idx_m, lab_m, cbuf, obuf,
             gsem0, ssem0, msem0):
    rows_per_w = lab_v.shape[0]
    n_vec = rows_per_w // _L
    wid = lax.axis_index("s") * _NC + lax.axis_index("c")
    base = wid * rows_per_w

    pltpu.sync_copy(lab_hbm.at[pl.ds(base, rows_per_w)], lab_v)
    pltpu.sync_copy(rand_hbm.at[pl.ds(base, rows_per_w)], rand_v)
    pltpu.sync_copy(ratio_hbm, ratio_v)
    for j in range(_W):
        pltpu.sync_copy(zrow_hbm, obuf.at[pl.ds(j * _G, _G)])

    ratio = ratio_v[...]
    iota = _iota16()
    ones_f = jnp.full((_L,), 1.0, jnp.float32)
    zeros_f = jnp.full((_L,), 0.0, jnp.float32)

    def compact(i, cnt):
        cnt_u, cnt_m = cnt
        lv = lab_v[pl.ds(i * _L, _L)]
        rv = rand_v[pl.ds(i * _L, _L)]
        m = (rv < ratio) & (lv >= _splat(0))
        mi = jnp.where(m, _splat(1), _splat(0))
        ids = _splat(base) + _splat(i * _L) + iota
        pos_u = _splat(cnt_u) + plsc.cumsum(_splat(1) - mi) - _splat(1)
        plsc.store_scatter(idx_u, [pos_u // _splat(_G), pos_u % _splat(_G)],
                           ids, mask=~m)
        pos_m = _splat(cnt_m) + plsc.cumsum(mi) - _splat(1)
        plsc.store_scatter(idx_m, [pos_m // _splat(_G), pos_m % _splat(_G)],
                           ids, mask=m)
        plsc.store_scatter(lab_m, [pos_m // _splat(_G), pos_m % _splat(_G)],
                           lv, mask=m)
        return cnt_u + jnp.sum(_splat(1) - mi), cnt_m + jnp.sum(mi)

    k_u, k_m = lax.fori_loop(0, n_vec, compact, (jnp.int32(0), jnp.int32(0)))

    def pad_tail(idx2d, k, also=None):
        rem = k % _G

        @pl.when((k > 0) & (rem != 0))
        def _():
            g = k // _G
            msk = iota >= _splat(rem)
            first = plsc.load_gather(idx2d, [_splat(0), _splat(0)])
            plsc.store_scatter(idx2d, [_splat(g), iota], first, mask=msk)
            if also is not None:
                first2 = plsc.load_gather(also, [_splat(0), _splat(0)])
                plsc.store_scatter(also, [_splat(g), iota], first2, mask=msk)

    pad_tail(idx_u, k_u)
    pad_tail(idx_m, k_m, also=lab_m)

    q_u = (k_u + (_G - 1)) // _G
    q_m = (k_m + (_G - 1)) // _G
    n_win = (jnp.maximum(q_u, q_m) + (_W - 1)) // _W

    def onehot_cols(g):
        lv0 = plsc.load_gather(lab_m, [_splat(g), iota])
        return jnp.clip(lv0, 0, 1023)

    def window(w, carry):
        g0 = w * _W

        for j in range(_W):
            @pl.when(g0 + j < q_u)
            def _(j=j):
                pltpu.async_copy(pred_hbm.at[idx_u.at[g0 + j]],
                                 cbuf.at[pl.ds(j * _G, _G)], gsem0)

        for j in range(_W):
            @pl.when(g0 + j < q_m)
            def _(j=j):
                oslab = obuf.at[pl.ds(j * _G, _G)]
                lv0 = onehot_cols(g0 + j)
                plsc.store_scatter(oslab, [iota, lv0], ones_f)
                pltpu.async_copy(oslab, out_hbm.at[idx_m.at[g0 + j]], msem0)

        for j in range(_W):
            @pl.when(g0 + j < q_u)
            def _(j=j):
                cslab = cbuf.at[pl.ds(j * _G, _G)]
                pltpu.make_async_copy(pred_hbm.at[idx_u.at[g0 + j]],
                                      cslab, gsem0).wait()
                pltpu.async_copy(cslab, out_hbm.at[idx_u.at[g0 + j]], ssem0)

        for j in range(_W):
            @pl.when(g0 + j < q_m)
            def _(j=j):
                oslab = obuf.at[pl.ds(j * _G, _G)]
                pltpu.make_async_copy(oslab, out_hbm.at[idx_m.at[g0 + j]],
                                      msem0).wait()
                lv0 = onehot_cols(g0 + j)
                plsc.store_scatter(oslab, [iota, lv0], zeros_f)

        for j in range(_W):
            @pl.when(g0 + j < q_u)
            def _(j=j):
                pltpu.make_async_copy(cbuf.at[pl.ds(j * _G, _G)],
                                      out_hbm.at[idx_u.at[g0 + j]],
                                      ssem0).wait()
        return carry

    lax.fori_loop(0, n_win, window, jnp.int32(0))


def kernel(obj_sem_cls_pred, obj_labels, cur_step, total_steps):
    b, n, c = obj_sem_cls_pred.shape
    rows = b * n
    rows_per_w = rows // _NW
    n_gran = (rows_per_w + _G - 1) // _G + 1
    mixup_ratio = jnp.clip(
        (total_steps * STAGE2_RATE - cur_step)
        / ((STAGE2_RATE - STAGE1_RATE) * total_steps),
        0.0,
        1.0,
    ).astype(jnp.float32)
    random_numer = jax.random.uniform(
        jax.random.key(42), (b, n), dtype=jnp.float32
    )

    pred2d = obj_sem_cls_pred.reshape(rows, c)
    lab1d = obj_labels.astype(jnp.int32).reshape(rows)
    rand1d = random_numer.reshape(rows)
    ratio16 = jnp.full((_L,), mixup_ratio, jnp.float32)
    zrow = jnp.zeros((_G, c), jnp.float32)

    mesh = plsc.VectorSubcoreMesh(
        core_axis_name="c", subcore_axis_name="s",
        num_cores=_NC, num_subcores=_NS,
    )
    run = functools.partial(
        pl.kernel,
        out_type=jax.ShapeDtypeStruct((rows, c), jnp.float32),
        mesh=mesh,
        compiler_params=pltpu.CompilerParams(needs_layout_passes=False),
        scratch_types=[
            pltpu.VMEM((rows_per_w,), jnp.int32),
            pltpu.VMEM((rows_per_w,), jnp.float32),
            pltpu.VMEM((_L,), jnp.float32),
            pltpu.VMEM((n_gran, _G), jnp.int32),
            pltpu.VMEM((n_gran, _G), jnp.int32),
            pltpu.VMEM((n_gran, _G), jnp.int32),
            pltpu.VMEM((_W * _G, c), jnp.float32),
            pltpu.VMEM((_W * _G, c), jnp.float32),
            pltpu.SemaphoreType.DMA,
            pltpu.SemaphoreType.DMA,
            pltpu.SemaphoreType.DMA,
        ],
    )(_sc_body)
    out = run(pred2d, lab1d, rand1d, ratio16, zrow)
    return out.reshape(b, n, c)


# speedup vs baseline: 1.0430x; 1.0315x over previous
import functools

import jax
import jax.numpy as jnp
from jax import lax
from jax.experimental import pallas as pl
from jax.experimental.pallas import tpu as pltpu
from jax.experimental.pallas import tpu_sc as plsc

STAGE1_RATE = 0.5
STAGE2_RATE = 0.75

_NC = 2
_NS = 16
_L = 16
_NW = _NC * _NS


def _iota16():
    return lax.iota(jnp.int32, _L)


def _splat(x, dtype=jnp.int32):
    return jnp.full((_L,), x, dtype)


def _idx_row(ref2d, g):
    return plsc.load_gather(ref2d, [_splat(g), _iota16()])


def _sc_body(pred_hbm, lab_hbm, rand_hbm, ratio_hbm, zrow_hbm, out_hbm,
             lab_v, rand_v, ratio_v, idx_u, idx_m, lab_m, cbuf, obuf,
             gsem, ssem, msem):
    rows_per_w = lab_v.shape[0]
    n_vec = rows_per_w // _L
    wid = lax.axis_index("s") * _NC + lax.axis_index("c")
    base = wid * rows_per_w

    pltpu.sync_copy(lab_hbm.at[pl.ds(base, rows_per_w)], lab_v)
    pltpu.sync_copy(rand_hbm.at[pl.ds(base, rows_per_w)], rand_v)
    pltpu.sync_copy(ratio_hbm, ratio_v)
    pltpu.sync_copy(zrow_hbm, obuf.at[pl.ds(0, _L)])
    pltpu.sync_copy(zrow_hbm, obuf.at[pl.ds(_L, _L)])

    ratio = ratio_v[...]
    iota = _iota16()
    ones_f = jnp.full((_L,), 1.0, jnp.float32)
    zeros_f = jnp.full((_L,), 0.0, jnp.float32)

    def compact(i, cnt):
        cnt_u, cnt_m = cnt
        lv = lab_v[pl.ds(i * _L, _L)]
        rv = rand_v[pl.ds(i * _L, _L)]
        m = (rv < ratio) & (lv >= _splat(0))
        mi = jnp.where(m, _splat(1), _splat(0))
        ids = _splat(base) + _splat(i * _L) + iota
        pos_u = _splat(cnt_u) + plsc.cumsum(_splat(1) - mi) - _splat(1)
        plsc.store_scatter(idx_u, [pos_u >> 4, pos_u & 15], ids, mask=~m)
        pos_m = _splat(cnt_m) + plsc.cumsum(mi) - _splat(1)
        plsc.store_scatter(idx_m, [pos_m >> 4, pos_m & 15], ids, mask=m)
        plsc.store_scatter(lab_m, [pos_m >> 4, pos_m & 15], lv, mask=m)
        return cnt_u + jnp.sum(_splat(1) - mi), cnt_m + jnp.sum(mi)

    k_u, k_m = lax.fori_loop(0, n_vec, compact, (jnp.int32(0), jnp.int32(0)))

    def pad_tail(idx2d, k, also=None):
        rem = k & 15

        @pl.when((k > 0) & (rem != 0))
        def _():
            g = k >> 4
            keep = iota < _splat(rem)
            first = plsc.load_gather(idx2d, [_splat(0), _splat(0)])
            plsc.store_scatter(idx2d, [_splat(g), iota], first, mask=~keep)
            if also is not None:
                first2 = plsc.load_gather(also, [_splat(0), _splat(0)])
                plsc.store_scatter(also, [_splat(g), iota], first2, mask=~keep)

    pad_tail(idx_u, k_u)
    pad_tail(idx_m, k_m, also=lab_m)

    q_u = (k_u + 15) >> 4
    q_m = (k_m + 15) >> 4
    n_win = jnp.maximum((q_u + 1) >> 1, (q_m + 1) >> 1)

    def window(w, carry):
        g0 = w * 2

        for j in range(2):
            @pl.when(g0 + j < q_u)
            def _(j=j):
                idxv = _idx_row(idx_u, g0 + j)
                pltpu.async_copy(pred_hbm.at[idxv],
                                 cbuf.at[pl.ds(j * _L, _L)], gsem)

        for j in range(2):
            @pl.when(g0 + j < q_m)
            def _(j=j):
                labv = jnp.clip(_idx_row(lab_m, g0 + j), 0, 1023)
                rowsv = _splat(j * _L) + iota
                plsc.store_scatter(obuf, [rowsv, labv], ones_f)
                idxv = _idx_row(idx_m, g0 + j)
                pltpu.async_copy(obuf.at[pl.ds(j * _L, _L)],
                                 out_hbm.at[idxv], msem)

        for j in range(2):
            @pl.when(g0 + j < q_u)
            def _(j=j):
                idxv = _idx_row(idx_u, g0 + j)
                pltpu.make_async_copy(pred_hbm.at[idxv],
                                      cbuf.at[pl.ds(j * _L, _L)], gsem).wait()
                pltpu.async_copy(cbuf.at[pl.ds(j * _L, _L)],
                                 out_hbm.at[idxv], ssem)

        for j in range(2):
            @pl.when(g0 + j < q_m)
            def _(j=j):
                idxv = _idx_row(idx_m, g0 + j)
                pltpu.make_async_copy(obuf.at[pl.ds(j * _L, _L)],
                                      out_hbm.at[idxv], msem).wait()
                labv = jnp.clip(_idx_row(lab_m, g0 + j), 0, 1023)
                rowsv = _splat(j * _L) + iota
                plsc.store_scatter(obuf, [rowsv, labv], zeros_f)

        for j in range(2):
            @pl.when(g0 + j < q_u)
            def _(j=j):
                idxv = _idx_row(idx_u, g0 + j)
                pltpu.make_async_copy(cbuf.at[pl.ds(j * _L, _L)],
                                      out_hbm.at[idxv], ssem).wait()
        return carry

    lax.fori_loop(0, n_win, window, jnp.int32(0))


def kernel(obj_sem_cls_pred, obj_labels, cur_step, total_steps):
    b, n, c = obj_sem_cls_pred.shape
    rows = b * n
    rows_per_w = rows // _NW
    mixup_ratio = jnp.clip(
        (total_steps * STAGE2_RATE - cur_step)
        / ((STAGE2_RATE - STAGE1_RATE) * total_steps),
        0.0,
        1.0,
    ).astype(jnp.float32)
    random_numer = jax.random.uniform(
        jax.random.key(42), (b, n), dtype=jnp.float32
    )

    pred2d = obj_sem_cls_pred.reshape(rows, c)
    lab1d = obj_labels.astype(jnp.int32).reshape(rows)
    rand1d = random_numer.reshape(rows)
    ratio16 = jnp.full((_L,), mixup_ratio, jnp.float32)
    zrow = jnp.zeros((_L, c), jnp.float32)

    mesh = plsc.VectorSubcoreMesh(
        core_axis_name="c", subcore_axis_name="s",
        num_cores=_NC, num_subcores=_NS,
    )
    run = functools.partial(
        pl.kernel,
        out_type=jax.ShapeDtypeStruct((rows, c), jnp.float32),
        mesh=mesh,
        compiler_params=pltpu.CompilerParams(needs_layout_passes=False),
        scratch_types=[
            pltpu.VMEM((rows_per_w,), jnp.int32),
            pltpu.VMEM((rows_per_w,), jnp.float32),
            pltpu.VMEM((_L,), jnp.float32),
            pltpu.VMEM((rows_per_w // _L, _L), jnp.int32),
            pltpu.VMEM((rows_per_w // _L, _L), jnp.int32),
            pltpu.VMEM((rows_per_w // _L, _L), jnp.int32),
            pltpu.VMEM((2 * _L, c), jnp.float32),
            pltpu.VMEM((2 * _L, c), jnp.float32),
            pltpu.SemaphoreType.DMA,
            pltpu.SemaphoreType.DMA,
            pltpu.SemaphoreType.DMA,
        ],
    )(_sc_body)
    out = run(pred2d, lab1d, rand1d, ratio16, zrow)
    return out.reshape(b, n, c)
